# Initial kernel scaffold; baseline (speedup 1.0000x reference)
import jax
import jax.numpy as jnp
from jax.experimental import pallas as pl


def kernel(x, fgl_v, fgl_g, fgl_b, fc_w, fc_b):
    raise NotImplementedError("write your pallas kernel here")


# speedup vs baseline: 15.9764x; 15.9764x over previous
import jax
import jax.numpy as jnp
from jax.experimental import pallas as pl

_S = 256
_H = 128
_NC = 10
_BB = 32


def _body(x_ref, m_ref, cb_ref, out_ref):
    xb = x_ref[...]
    tl = jnp.sum(xb[:, :_H, :_H], axis=(1, 2))
    bl = jnp.sum(xb[:, _H:, :_H], axis=(1, 2))
    br = jnp.sum(xb[:, _H:, _H:], axis=(1, 2))
    tr = jnp.sum(xb[:, :_H, _H:], axis=(1, 2))
    m = m_ref[...]
    out_ref[...] = (tl[:, None] * m[0][None, :]
                    + bl[:, None] * m[1][None, :]
                    + br[:, None] * m[2][None, :]
                    + tr[:, None] * m[3][None, :]
                    + cb_ref[...])


def kernel(x, fgl_v, fgl_g, fgl_b, fc_w, fc_b):
    n = x.shape[0]
    vnorm = jnp.sqrt(jnp.sum(fgl_v ** 2, axis=(1, 2), keepdims=True))
    w = (fgl_g * fgl_v / vnorm).reshape(4, 4)
    fc_w3 = fc_w.reshape(_NC, 4, 4)
    m = jnp.einsum("ij,cij->ic", w, fc_w3)
    cb = (fc_b + jnp.einsum("ij,cij->c", fgl_b, fc_w3)).reshape(1, _NC)
    return pl.pallas_call(
        _body,
        grid=(n // _BB,),
        in_specs=[
            pl.BlockSpec((_BB, _S, _S), lambda i: (i, 0, 0)),
            pl.BlockSpec((4, _NC), lambda i: (0, 0)),
            pl.BlockSpec((1, _NC), lambda i: (0, 0)),
        ],
        out_specs=pl.BlockSpec((_BB, _NC), lambda i: (i, 0)),
        out_shape=jax.ShapeDtypeStruct((n, _NC), jnp.float32),
    )(x, m, cb)
